# baseline (speedup 1.0000x reference)
import functools

import jax
import jax.numpy as jnp
from jax import lax
from jax.experimental import pallas as pl
from jax.experimental.pallas import tpu as pltpu
from jax.experimental.pallas import tpu_sc as plsc

MAX_V = 1000
VPAD = 1024
D = 128
NB = 1024
L = 50
NW = 32
RPW = NB // NW


def _table_body(w_ref, b_ref, o_ref):
    iota_v = lax.broadcasted_iota(jnp.int32, (VPAD, MAX_V), 0)
    iota_c = lax.broadcasted_iota(jnp.int32, (VPAD, MAX_V), 1)
    onehot = (iota_v == iota_c).astype(jnp.float32)
    o_ref[...] = (
        lax.dot_general(
            onehot,
            w_ref[...],
            dimension_numbers=(((1,), (1,)), ((), ())),
            preferred_element_type=jnp.float32,
        )
        + b_ref[...]
    )


def _build_table(W, b_row):
    return pl.pallas_call(
        _table_body,
        out_shape=jax.ShapeDtypeStruct((VPAD, D), jnp.float32),
    )(W, b_row)


_mesh = plsc.VectorSubcoreMesh(core_axis_name="c", subcore_axis_name="s")


@functools.partial(
    pl.kernel,
    mesh=_mesh,
    out_type=jax.ShapeDtypeStruct((NB, L, D), jnp.float32),
    scratch_types=[
        pltpu.VMEM((RPW, L), jnp.int32),
        pltpu.VMEM((2, 4, L, D), jnp.float32),
        pltpu.VMEM_SHARED((VPAD, D), jnp.float32),
        pltpu.SemaphoreType.DMA,
        pltpu.SemaphoreType.DMA,
    ],
)
def _sc_emb(table_hbm, x_hbm, out_hbm, idx_v, buf, tbl_s, gsem, ssem):
    c = lax.axis_index("c")
    s = lax.axis_index("s")
    wid = s * 2 + c
    r0 = wid * RPW
    pltpu.sync_copy(x_hbm.at[pl.ds(r0, RPW)], idx_v)
    @pl.when(s == 0)
    def _():
        pltpu.sync_copy(table_hbm, tbl_s)
    plsc.subcore_barrier()

    nquads = RPW // 4
    gath = [None] * nquads
    writ = [None] * nquads

    def issue_gather(q):
        k = q % 2
        return tuple(
            pltpu.async_copy(
                tbl_s.at[idx_v.at[4 * q + j]], buf.at[k, j], gsem
            )
            for j in range(4)
        )

    gath[0] = issue_gather(0)
    gath[1] = issue_gather(1)
    for q in range(nquads):
        k = q % 2
        if q >= 2:
            writ[q - 2].wait()
        for cp in gath[q]:
            cp.wait()
        writ[q] = pltpu.async_copy(
            buf.at[k], out_hbm.at[pl.ds(r0 + 4 * q, 4)], ssem
        )
        if q + 2 < nquads:
            gath[q + 2] = issue_gather(q + 2)
    writ[nquads - 2].wait()
    writ[nquads - 1].wait()


def kernel(x, W, b):
    table = _build_table(W, b.reshape(1, D))
    return _sc_emb(table, x.astype(jnp.int32))


# speedup vs baseline: 1.0067x; 1.0067x over previous
import functools

import jax
import jax.numpy as jnp
from jax import lax
from jax.experimental import pallas as pl
from jax.experimental.pallas import tpu as pltpu
from jax.experimental.pallas import tpu_sc as plsc

MAX_V = 1000
VPAD = 1024
D = 128
NB = 1024
L = 50
NW = 32
RPW = NB // NW


def _table_body(w_ref, b_ref, o_ref):
    iota_v = lax.broadcasted_iota(jnp.int32, (VPAD, MAX_V), 0)
    iota_c = lax.broadcasted_iota(jnp.int32, (VPAD, MAX_V), 1)
    onehot = (iota_v == iota_c).astype(jnp.float32)
    o_ref[...] = (
        lax.dot_general(
            onehot,
            w_ref[...],
            dimension_numbers=(((1,), (1,)), ((), ())),
            preferred_element_type=jnp.float32,
        )
        + b_ref[...]
    )


def _build_table(W, b_row):
    return pl.pallas_call(
        _table_body,
        out_shape=jax.ShapeDtypeStruct((VPAD, D), jnp.float32),
    )(W, b_row)


_mesh = plsc.VectorSubcoreMesh(core_axis_name="c", subcore_axis_name="s")


@functools.partial(
    pl.kernel,
    mesh=_mesh,
    out_type=jax.ShapeDtypeStruct((NB, L, D), jnp.float32),
    scratch_types=[
        pltpu.VMEM((RPW, L), jnp.int32),
        pltpu.VMEM((3, 2, L, D), jnp.float32),
        pltpu.VMEM_SHARED((VPAD, D), jnp.float32),
        pltpu.SemaphoreType.DMA,
        pltpu.SemaphoreType.DMA,
    ],
)
def _sc_emb(table_hbm, x_hbm, out_hbm, idx_v, buf, tbl_s, gsem, ssem):
    c = lax.axis_index("c")
    s = lax.axis_index("s")
    wid = s * 2 + c
    r0 = wid * RPW
    pltpu.sync_copy(x_hbm.at[pl.ds(r0, RPW)], idx_v)
    stripe = VPAD // 16
    pltpu.sync_copy(
        table_hbm.at[pl.ds(s * stripe, stripe)],
        tbl_s.at[pl.ds(s * stripe, stripe)],
    )
    plsc.subcore_barrier()

    npairs = RPW // 2
    gath = [None] * npairs
    writ = [None] * npairs

    def issue_gather(p):
        k = p % 3
        return (
            pltpu.async_copy(tbl_s.at[idx_v.at[2 * p]], buf.at[k, 0], gsem),
            pltpu.async_copy(tbl_s.at[idx_v.at[2 * p + 1]], buf.at[k, 1], gsem),
        )

    gath[0] = issue_gather(0)
    gath[1] = issue_gather(1)
    gath[2] = issue_gather(2)
    for p in range(npairs):
        k = p % 3
        if p >= 3:
            writ[p - 3].wait()
        gath[p][0].wait()
        gath[p][1].wait()
        writ[p] = pltpu.async_copy(
            buf.at[k], out_hbm.at[pl.ds(r0 + 2 * p, 2)], ssem
        )
        if p + 3 < npairs:
            gath[p + 3] = issue_gather(p + 3)
    writ[npairs - 3].wait()
    writ[npairs - 2].wait()
    writ[npairs - 1].wait()


def kernel(x, W, b):
    table = _build_table(W, b.reshape(1, D))
    return _sc_emb(table, x.astype(jnp.int32))


# speedup vs baseline: 1.0232x; 1.0163x over previous
import functools

import jax
import jax.numpy as jnp
from jax import lax
from jax.experimental import pallas as pl
from jax.experimental.pallas import tpu as pltpu
from jax.experimental.pallas import tpu_sc as plsc

MAX_V = 1000
VPAD = 1024
D = 128
NB = 1024
L = 50
NW = 32
RPW = NB // NW


def _table_body(w_ref, b_ref, o_ref):
    iota_v = lax.broadcasted_iota(jnp.int32, (VPAD, MAX_V), 0)
    iota_c = lax.broadcasted_iota(jnp.int32, (VPAD, MAX_V), 1)
    onehot = (iota_v == iota_c).astype(jnp.float32)
    o_ref[...] = (
        lax.dot_general(
            onehot,
            w_ref[...],
            dimension_numbers=(((1,), (1,)), ((), ())),
            preferred_element_type=jnp.float32,
        )
        + b_ref[...]
    )


def _build_table(W, b_row):
    return pl.pallas_call(
        _table_body,
        out_shape=jax.ShapeDtypeStruct((VPAD, D), jnp.float32),
    )(W, b_row)


_mesh = plsc.VectorSubcoreMesh(core_axis_name="c", subcore_axis_name="s")


@functools.partial(
    pl.kernel,
    mesh=_mesh,
    out_type=jax.ShapeDtypeStruct((NB, L, D), jnp.float32),
    scratch_types=[
        pltpu.VMEM((RPW, L), jnp.int32),
        pltpu.VMEM((3, 2, L, D), jnp.float32),
        pltpu.VMEM_SHARED((VPAD, D), jnp.float32),
        pltpu.SemaphoreType.DMA,
        pltpu.SemaphoreType.DMA,
    ],
)
def _sc_emb(table_hbm, x_hbm, out_hbm, idx_v, buf, tbl_s, gsem, ssem):
    c = lax.axis_index("c")
    s = lax.axis_index("s")
    wid = s * 2 + c
    r0 = wid * RPW
    stripe = VPAD // 16
    tcp = pltpu.async_copy(
        table_hbm.at[pl.ds(s * stripe, stripe)],
        tbl_s.at[pl.ds(s * stripe, stripe)],
        ssem,
    )
    pltpu.sync_copy(x_hbm.at[pl.ds(r0, RPW)], idx_v)
    tcp.wait()
    plsc.subcore_barrier()

    npairs = RPW // 2
    gath = [None] * npairs
    writ = [None] * npairs

    def issue_gather(p):
        k = p % 3
        return (
            pltpu.async_copy(tbl_s.at[idx_v.at[2 * p]], buf.at[k, 0], gsem),
            pltpu.async_copy(tbl_s.at[idx_v.at[2 * p + 1]], buf.at[k, 1], gsem),
        )

    gath[0] = issue_gather(0)
    gath[1] = issue_gather(1)
    gath[2] = issue_gather(2)
    for p in range(npairs):
        k = p % 3
        if p >= 3:
            writ[p - 3].wait()
        gath[p][0].wait()
        gath[p][1].wait()
        writ[p] = pltpu.async_copy(
            buf.at[k], out_hbm.at[pl.ds(r0 + 2 * p, 2)], ssem
        )
        if p + 3 < npairs:
            gath[p + 3] = issue_gather(p + 3)
    writ[npairs - 3].wait()
    writ[npairs - 2].wait()
    writ[npairs - 1].wait()


def kernel(x, W, b):
    table = _build_table(W, b.reshape(1, D))
    return _sc_emb(table, x.astype(jnp.int32))
